# Initial kernel scaffold; baseline (speedup 1.0000x reference)
import jax
import jax.numpy as jnp
from jax.experimental import pallas as pl


def kernel(x, node_type, item_id, emb_weight, W, b):
    raise NotImplementedError("write your pallas kernel here")


# speedup vs baseline: 2.3131x; 2.3131x over previous
import jax
import jax.numpy as jnp
from jax.experimental import pallas as pl
from jax.experimental.pallas import tpu as pltpu

_N = 100000
_CH = 128
_NT = 16
_B = 2000
_G = _N // _B


def _body(item_ref, nt_ref, x_ref, emb_ref, wt_ref, b_ref, out_ref):
    nt = nt_ref[0]
    x = x_ref[...]
    lin = jnp.dot(x, wt_ref[...], preferred_element_type=jnp.float32) + b_ref[0, :]
    oh = (nt == jax.lax.broadcasted_iota(jnp.int32, (_B, _NT), 1)
          ).astype(jnp.float32)
    emb_rows = jnp.dot(oh, emb_ref[...], preferred_element_type=jnp.float32)
    maskf = (nt == item_ref[0]).astype(jnp.float32)
    out_ref[...] = emb_rows + maskf * (lin - emb_rows)


def kernel(x, node_type, item_id, emb_weight, W, b):
    wt = W.T
    nt3 = node_type.reshape(_G, _B, 1)
    item = jnp.asarray(item_id, jnp.int32).reshape(1)
    b2 = b.reshape(1, _CH)
    return pl.pallas_call(
        _body,
        grid=(_G,),
        in_specs=[
            pl.BlockSpec(memory_space=pltpu.SMEM),
            pl.BlockSpec((1, _B, 1), lambda i: (i, 0, 0)),
            pl.BlockSpec((_B, _CH), lambda i: (i, 0)),
            pl.BlockSpec((_NT, _CH), lambda i: (0, 0)),
            pl.BlockSpec((_CH, _CH), lambda i: (0, 0)),
            pl.BlockSpec((1, _CH), lambda i: (0, 0)),
        ],
        out_specs=pl.BlockSpec((_B, _CH), lambda i: (i, 0)),
        out_shape=jax.ShapeDtypeStruct((_N, _CH), jnp.float32),
        compiler_params=pltpu.CompilerParams(
            dimension_semantics=("arbitrary",),
        ),
    )(item, nt3, x, emb_weight, wt, b2)
